# baseline (speedup 1.0000x reference)
import functools

import jax
import jax.numpy as jnp
from jax import lax
from jax.experimental import pallas as pl
from jax.experimental.pallas import tpu as pltpu
from jax.experimental.pallas import tpu_sc as plsc

D = 1024
HALF = D // 2
ROPE_BASE = 10000.0
LN_BASE = 9.210340371976184

NC = 2
NS = 16
NW = NC * NS

BATCH = 4
SEG = 64
SEQ = 4096
STRIDE = SEG + SEQ
TOK = BATCH * SEQ
ROWS_OUT = BATCH * STRIDE
TOK_PER_W = TOK // NW
CHUNK = 32
NCHUNK = TOK_PER_W // CHUNK
CLS_PER_W = (BATCH * SEG) // NW


NBUF = 3
CGRP = 8


def _sc_body(ids_hbm, table_hbm, type_hbm, cls_hbm, out_hbm,
             idx_v, b0, b1, b2, type_v,
             g0, g1, g2, s0, s1, s2):
    rows_b = (b0, b1, b2)
    gsem = (g0, g1, g2)
    ssem = (s0, s1, s2)
    wid = lax.axis_index("s") * NC + lax.axis_index("c")
    batch = wid // (NW // BATCH)

    pltpu.sync_copy(type_hbm.at[1], type_v)
    ids_base = pl.multiple_of(wid * TOK_PER_W, TOK_PER_W)
    pltpu.sync_copy(ids_hbm.at[pl.ds(ids_base, TOK_PER_W)], idx_v)

    cls_src = pl.multiple_of(wid * CLS_PER_W, CLS_PER_W)
    cls_dst = pl.multiple_of(
        batch * STRIDE + (wid % (NW // BATCH)) * CLS_PER_W, CLS_PER_W)
    pltpu.sync_copy(cls_hbm.at[pl.ds(cls_src, CLS_PER_W)],
                    b0.at[pl.ds(0, CLS_PER_W)])
    pltpu.sync_copy(b0.at[pl.ds(0, CLS_PER_W)],
                    out_hbm.at[pl.ds(cls_dst, CLS_PER_W)])

    out_base = batch * STRIDE + SEG + (ids_base - batch * SEQ)

    def start_gather(c, b):
        pltpu.async_copy(table_hbm.at[idx_v.at[pl.ds(c * CHUNK, CHUNK)]],
                         rows_b[b], gsem[b])

    def start_scatter(c, b):
        obase = pl.multiple_of(out_base + c * CHUNK, CHUNK)
        pltpu.async_copy(rows_b[b], out_hbm.at[pl.ds(obase, CHUNK)], ssem[b])

    def wait_gather(b):
        pltpu.make_async_copy(table_hbm.at[idx_v.at[pl.ds(0, CHUNK)]],
                              rows_b[b], gsem[b]).wait()

    def wait_scatter(b):
        pltpu.make_async_copy(rows_b[b],
                              out_hbm.at[pl.ds(out_base, CHUNK)],
                              ssem[b]).wait()

    def add_type(b):
        rows = rows_b[b]
        for grp in range(D // 16 // CGRP):
            tvs = [type_v[pl.ds((grp * CGRP + j) * 16, 16)]
                   for j in range(CGRP)]

            def row_body(r, carry, _grp=grp, _tvs=tvs):
                for j in range(CGRP):
                    sl = pl.ds((_grp * CGRP + j) * 16, 16)
                    rows[r, sl] = rows[r, sl] + _tvs[j]
                return carry

            lax.fori_loop(0, CHUNK, row_body, 0)

    start_gather(0, 0)
    for c in range(NCHUNK):
        b = c % NBUF
        if c + 1 < NCHUNK:
            bn = (c + 1) % NBUF
            if c + 1 >= NBUF:
                wait_scatter(bn)
            start_gather(c + 1, bn)
        wait_gather(b)
        add_type(b)
        start_scatter(c, b)

    for c in range(NCHUNK - NBUF, NCHUNK):
        wait_scatter(c % NBUF)


@functools.cache
def _sc_gather():
    return pl.kernel(
        _sc_body,
        out_type=jax.ShapeDtypeStruct((ROWS_OUT, D), jnp.float32),
        mesh=plsc.VectorSubcoreMesh(core_axis_name="c", subcore_axis_name="s",
                                    num_cores=NC, num_subcores=NS),
        scratch_types=(
            [pltpu.VMEM((TOK_PER_W,), jnp.int32)]
            + [pltpu.VMEM((CHUNK, D), jnp.float32) for _ in range(NBUF)]
            + [pltpu.VMEM((D,), jnp.float32)]
            + [pltpu.SemaphoreType.DMA for _ in range(2 * NBUF)]
        ),
    )


def _tc_body(cls_ref, w_ref, b_ref, t0_ref, clsout_ref, cos_ref, sin_ref,
             cb, sb):
    a = pl.program_id(0)

    @pl.when(a < BATCH)
    def _():
        acc = lax.dot_general(cls_ref[...], w_ref[...],
                              (((1,), (1,)), ((), ())),
                              preferred_element_type=jnp.float32)
        clsout_ref[...] = acc + b_ref[...] + t0_ref[...]

    @pl.when(a == 0)
    def _():
        r = lax.broadcasted_iota(jnp.int32, (SEG, HALF), 0).astype(jnp.float32)
        k = lax.broadcasted_iota(jnp.int32, (SEG, HALF), 1).astype(jnp.float32)
        inv = jnp.exp(k * (-LN_BASE / HALF))
        f = r * inv
        cb[...] = jnp.cos(f)
        sb[...] = jnp.sin(f)

    pb = jnp.maximum(a - 1, 0).astype(jnp.float32) * float(SEG)
    k1 = lax.broadcasted_iota(jnp.int32, (1, HALF), 1).astype(jnp.float32)
    fA = pb * jnp.exp(k1 * (-LN_BASE / HALF))
    cA = jnp.cos(fA)
    sA = jnp.sin(fA)
    cos_half = cA * cb[...] - sA * sb[...]
    sin_half = sA * cb[...] + cA * sb[...]
    cos_ref[:, 0:HALF] = cos_half
    cos_ref[:, HALF:D] = cos_half
    sin_ref[:, 0:HALF] = sin_half
    sin_ref[:, HALF:D] = sin_half


def _tc_all(cls2d, w, b2d, t02d):
    nblk = STRIDE // SEG
    return pl.pallas_call(
        _tc_body,
        grid=(nblk,),
        in_specs=[
            pl.BlockSpec((SEG, D), lambda a: (jnp.minimum(a, BATCH - 1), 0)),
            pl.BlockSpec((D, D), lambda a: (0, 0)),
            pl.BlockSpec((1, D), lambda a: (0, 0)),
            pl.BlockSpec((1, D), lambda a: (0, 0)),
        ],
        out_specs=[
            pl.BlockSpec((SEG, D), lambda a: (jnp.minimum(a, BATCH - 1), 0)),
            pl.BlockSpec((SEG, D), lambda a: (a, 0)),
            pl.BlockSpec((SEG, D), lambda a: (a, 0)),
        ],
        out_shape=[
            jax.ShapeDtypeStruct((BATCH * SEG, D), jnp.float32),
            jax.ShapeDtypeStruct((STRIDE, D), jnp.float32),
            jax.ShapeDtypeStruct((STRIDE, D), jnp.float32),
        ],
        scratch_shapes=[
            pltpu.VMEM((SEG, HALF), jnp.float32),
            pltpu.VMEM((SEG, HALF), jnp.float32),
        ],
    )(cls2d, w, b2d, t02d)


def kernel(cls_embeddings, token_ids, token_table, type_table, W, b):
    ids = token_ids.reshape(-1).astype(jnp.int32)
    cls2d = cls_embeddings.reshape(BATCH * SEG, D).astype(jnp.float32)
    cls_res, rope_cos, rope_sin = _tc_all(cls2d, W, b.reshape(1, D),
                                          type_table[0].reshape(1, D))
    out = _sc_gather()(ids, token_table, type_table, cls_res)
    return out.reshape(BATCH, STRIDE, D), rope_cos, rope_sin


# speedup vs baseline: 1.2589x; 1.2589x over previous
import functools

import jax
import jax.numpy as jnp
from jax import lax
from jax.experimental import pallas as pl
from jax.experimental.pallas import tpu as pltpu
from jax.experimental.pallas import tpu_sc as plsc

D = 1024
HALF = D // 2
ROPE_BASE = 10000.0
LN_BASE = 9.210340371976184

NC = 2
NS = 16
NW = NC * NS

BATCH = 4
SEG = 64
SEQ = 4096
STRIDE = SEG + SEQ
TOK = BATCH * SEQ
ROWS_OUT = BATCH * STRIDE
TOK_PER_W = TOK // NW
CHUNK = 32
NCHUNK = TOK_PER_W // CHUNK
CLS_PER_W = (BATCH * SEG) // NW


NBUF = 3
CGRP = 8


def _sc_body(ids_hbm, table_hbm, type_hbm, cls_hbm, out_hbm,
             idx_v, b0, b1, b2, type_v,
             g0, g1, g2, s0, s1, s2):
    rows_b = (b0, b1, b2)
    gsem = (g0, g1, g2)
    ssem = (s0, s1, s2)
    wid = lax.axis_index("s") * NC + lax.axis_index("c")
    batch = wid // (NW // BATCH)

    pltpu.sync_copy(type_hbm.at[1], type_v)
    ids_base = pl.multiple_of(wid * TOK_PER_W, TOK_PER_W)
    pltpu.sync_copy(ids_hbm.at[pl.ds(ids_base, TOK_PER_W)], idx_v)

    cls_src = pl.multiple_of(wid * CLS_PER_W, CLS_PER_W)
    cls_dst = pl.multiple_of(
        batch * STRIDE + (wid % (NW // BATCH)) * CLS_PER_W, CLS_PER_W)
    pltpu.sync_copy(cls_hbm.at[pl.ds(cls_src, CLS_PER_W)],
                    b0.at[pl.ds(0, CLS_PER_W)])
    pltpu.sync_copy(b0.at[pl.ds(0, CLS_PER_W)],
                    out_hbm.at[pl.ds(cls_dst, CLS_PER_W)])

    out_base = batch * STRIDE + SEG + (ids_base - batch * SEQ)

    def start_gather(c, b):
        pltpu.async_copy(table_hbm.at[idx_v.at[pl.ds(c * CHUNK, CHUNK)]],
                         rows_b[b], gsem[b])

    def start_scatter(c, b):
        obase = pl.multiple_of(out_base + c * CHUNK, CHUNK)
        pltpu.async_copy(rows_b[b], out_hbm.at[pl.ds(obase, CHUNK)], ssem[b])

    def wait_gather(b):
        pltpu.make_async_copy(table_hbm.at[idx_v.at[pl.ds(0, CHUNK)]],
                              rows_b[b], gsem[b]).wait()

    def wait_scatter(b):
        pltpu.make_async_copy(rows_b[b],
                              out_hbm.at[pl.ds(out_base, CHUNK)],
                              ssem[b]).wait()

    def add_type(b):
        rows = rows_b[b]
        for grp in range(D // 16 // CGRP):
            tvs = [type_v[pl.ds((grp * CGRP + j) * 16, 16)]
                   for j in range(CGRP)]

            def row_body(r, carry, _grp=grp, _tvs=tvs):
                for j in range(CGRP):
                    sl = pl.ds((_grp * CGRP + j) * 16, 16)
                    rows[r, sl] = rows[r, sl] + _tvs[j]
                return carry

            lax.fori_loop(0, CHUNK, row_body, 0)

    start_gather(0, 0)
    for c in range(NCHUNK):
        b = c % NBUF
        if c + 1 < NCHUNK:
            bn = (c + 1) % NBUF
            if c + 1 >= NBUF:
                wait_scatter(bn)
            start_gather(c + 1, bn)
        wait_gather(b)
        add_type(b)
        start_scatter(c, b)

    for c in range(NCHUNK - NBUF, NCHUNK):
        wait_scatter(c % NBUF)


@functools.cache
def _sc_gather():
    return pl.kernel(
        _sc_body,
        out_type=jax.ShapeDtypeStruct((ROWS_OUT, D), jnp.float32),
        mesh=plsc.VectorSubcoreMesh(core_axis_name="c", subcore_axis_name="s",
                                    num_cores=NC, num_subcores=NS),
        scratch_types=(
            [pltpu.VMEM((TOK_PER_W,), jnp.int32)]
            + [pltpu.VMEM((CHUNK, D), jnp.float32) for _ in range(NBUF)]
            + [pltpu.VMEM((D,), jnp.float32)]
            + [pltpu.SemaphoreType.DMA for _ in range(2 * NBUF)]
        ),
    )


def _cls_proj_body(cls_ref, w_ref, b_ref, t0_ref, o_ref):
    acc = lax.dot_general(cls_ref[...], w_ref[...],
                          (((1,), (1,)), ((), ())),
                          preferred_element_type=jnp.float32)
    o_ref[...] = acc + b_ref[...] + t0_ref[...]


def _cls_proj(cls2d, w, b2d, t02d):
    return pl.pallas_call(
        _cls_proj_body,
        out_shape=jax.ShapeDtypeStruct((BATCH * SEG, D), jnp.float32),
    )(cls2d, w, b2d, t02d)


def _rope_body(cos_ref, sin_ref, cb, sb):
    a = pl.program_id(0)

    @pl.when(a == 0)
    def _():
        r = lax.broadcasted_iota(jnp.int32, (SEG, HALF), 0).astype(jnp.float32)
        k = lax.broadcasted_iota(jnp.int32, (SEG, HALF), 1).astype(jnp.float32)
        inv = jnp.exp(k * (-LN_BASE / HALF))
        f = r * inv
        cb[...] = jnp.cos(f)
        sb[...] = jnp.sin(f)

    pb = jnp.maximum(a - 1, 0).astype(jnp.float32) * float(SEG)
    k1 = lax.broadcasted_iota(jnp.int32, (1, HALF), 1).astype(jnp.float32)
    fA = pb * jnp.exp(k1 * (-LN_BASE / HALF))
    cA = jnp.cos(fA)
    sA = jnp.sin(fA)
    cos_half = cA * cb[...] - sA * sb[...]
    sin_half = sA * cb[...] + cA * sb[...]
    cos_ref[:, 0:HALF] = cos_half
    cos_ref[:, HALF:D] = cos_half
    sin_ref[:, 0:HALF] = sin_half
    sin_ref[:, HALF:D] = sin_half


def _rope():
    nblk = STRIDE // SEG
    return pl.pallas_call(
        _rope_body,
        grid=(nblk,),
        out_specs=[
            pl.BlockSpec((SEG, D), lambda a: (a, 0)),
            pl.BlockSpec((SEG, D), lambda a: (a, 0)),
        ],
        out_shape=[
            jax.ShapeDtypeStruct((STRIDE, D), jnp.float32),
            jax.ShapeDtypeStruct((STRIDE, D), jnp.float32),
        ],
        scratch_shapes=[
            pltpu.VMEM((SEG, HALF), jnp.float32),
            pltpu.VMEM((SEG, HALF), jnp.float32),
        ],
    )()


def kernel(cls_embeddings, token_ids, token_table, type_table, W, b):
    ids = token_ids.reshape(-1).astype(jnp.int32)
    cls2d = cls_embeddings.reshape(BATCH * SEG, D).astype(jnp.float32)
    cls_res = _cls_proj(cls2d, W, b.reshape(1, D),
                        type_table[0].reshape(1, D))
    rope_cos, rope_sin = _rope()
    out = _sc_gather()(ids, token_table, type_table, cls_res)
    return out.reshape(BATCH, STRIDE, D), rope_cos, rope_sin


# speedup vs baseline: 1.3297x; 1.0562x over previous
import functools

import jax
import jax.numpy as jnp
from jax import lax
from jax.experimental import pallas as pl
from jax.experimental.pallas import tpu as pltpu
from jax.experimental.pallas import tpu_sc as plsc

D = 1024
HALF = D // 2
ROPE_BASE = 10000.0
LN_BASE = 9.210340371976184

NC = 2
NS = 16
NW = NC * NS

BATCH = 4
SEG = 64
SEQ = 4096
STRIDE = SEG + SEQ
TOK = BATCH * SEQ
ROWS_OUT = BATCH * STRIDE
TOK_PER_W = TOK // NW
CHUNK = 16
NCHUNK = TOK_PER_W // CHUNK
CLS_PER_W = (BATCH * SEG) // NW


NBUF = 4
CGRP = 8


def _sc_body(ids_hbm, table_hbm, type_hbm, out_hbm,
             idx_v, b0, b1, b2, b3, type_v,
             g0, g1, g2, g3, s0, s1, s2, s3):
    rows_b = (b0, b1, b2, b3)
    gsem = (g0, g1, g2, g3)
    ssem = (s0, s1, s2, s3)
    wid = lax.axis_index("s") * NC + lax.axis_index("c")
    batch = wid // (NW // BATCH)

    ids_base = pl.multiple_of(wid * TOK_PER_W, TOK_PER_W)
    pltpu.sync_copy(ids_hbm.at[pl.ds(ids_base, TOK_PER_W)], idx_v)

    out_base = batch * STRIDE + SEG + (ids_base - batch * SEQ)

    def start_gather(c, b):
        pltpu.async_copy(table_hbm.at[idx_v.at[pl.ds(c * CHUNK, CHUNK)]],
                         rows_b[b], gsem[b])

    def start_scatter(c, b):
        obase = pl.multiple_of(out_base + c * CHUNK, CHUNK)
        pltpu.async_copy(rows_b[b], out_hbm.at[pl.ds(obase, CHUNK)], ssem[b])

    def wait_gather(b):
        pltpu.make_async_copy(table_hbm.at[idx_v.at[pl.ds(0, CHUNK)]],
                              rows_b[b], gsem[b]).wait()

    def wait_scatter(b):
        pltpu.make_async_copy(rows_b[b],
                              out_hbm.at[pl.ds(out_base, CHUNK)],
                              ssem[b]).wait()

    def add_type(b):
        rows = rows_b[b]
        for grp in range(D // 16 // CGRP):
            tvs = [type_v[pl.ds((grp * CGRP + j) * 16, 16)]
                   for j in range(CGRP)]

            def row_body(r, carry, _grp=grp, _tvs=tvs):
                for j in range(CGRP):
                    sl = pl.ds((_grp * CGRP + j) * 16, 16)
                    rows[r, sl] = rows[r, sl] + _tvs[j]
                return carry

            lax.fori_loop(0, CHUNK, row_body, 0)

    start_gather(0, 0)
    start_gather(1, 1)
    pltpu.sync_copy(type_hbm.at[1], type_v)

    def step(i, carry):
        for b in range(NBUF):
            c = i * NBUF + b

            @pl.when(jnp.logical_and(c >= 2, c <= NCHUNK - 3))
            def _(b=b):
                wait_scatter((b + 2) % NBUF)

            @pl.when(c <= NCHUNK - 3)
            def _(c=c, b=b):
                start_gather(c + 2, (b + 2) % NBUF)

            wait_gather(b)
            add_type(b)
            start_scatter(c, b)
        return carry

    lax.fori_loop(0, NCHUNK // NBUF, step, 0)

    for b in range(NBUF):
        wait_scatter(b)


@functools.cache
def _sc_gather():
    return pl.kernel(
        _sc_body,
        out_type=jax.ShapeDtypeStruct((ROWS_OUT, D), jnp.float32),
        mesh=plsc.VectorSubcoreMesh(core_axis_name="c", subcore_axis_name="s",
                                    num_cores=NC, num_subcores=NS),
        scratch_types=(
            [pltpu.VMEM((TOK_PER_W,), jnp.int32)]
            + [pltpu.VMEM((CHUNK, D), jnp.float32) for _ in range(NBUF)]
            + [pltpu.VMEM((D,), jnp.float32)]
            + [pltpu.SemaphoreType.DMA for _ in range(2 * NBUF)]
        ),
    )


def _cls_proj_body(cls_ref, w_ref, b_ref, t0_ref, o_ref):
    acc = lax.dot_general(cls_ref[...], w_ref[...],
                          (((1,), (1,)), ((), ())),
                          preferred_element_type=jnp.float32)
    o_ref[...] = acc + b_ref[...] + t0_ref[...]


def _cls_proj(cls2d, w, b2d, t02d):
    return pl.pallas_call(
        _cls_proj_body,
        out_shape=jax.ShapeDtypeStruct((BATCH * SEG, D), jnp.float32),
    )(cls2d, w, b2d, t02d)


def _rope_body(cos_ref, sin_ref, cb, sb):
    a = pl.program_id(0)

    @pl.when(a == 0)
    def _():
        r = lax.broadcasted_iota(jnp.int32, (SEG, HALF), 0).astype(jnp.float32)
        k = lax.broadcasted_iota(jnp.int32, (SEG, HALF), 1).astype(jnp.float32)
        inv = jnp.exp(k * (-LN_BASE / HALF))
        f = r * inv
        cb[...] = jnp.cos(f)
        sb[...] = jnp.sin(f)

    pb = jnp.maximum(a - 1, 0).astype(jnp.float32) * float(SEG)
    k1 = lax.broadcasted_iota(jnp.int32, (1, HALF), 1).astype(jnp.float32)
    fA = pb * jnp.exp(k1 * (-LN_BASE / HALF))
    cA = jnp.cos(fA)
    sA = jnp.sin(fA)
    cos_half = cA * cb[...] - sA * sb[...]
    sin_half = sA * cb[...] + cA * sb[...]
    cos_ref[:, 0:HALF] = cos_half
    cos_ref[:, HALF:D] = cos_half
    sin_ref[:, 0:HALF] = sin_half
    sin_ref[:, HALF:D] = sin_half


def _rope():
    nblk = STRIDE // SEG
    return pl.pallas_call(
        _rope_body,
        grid=(nblk,),
        out_specs=[
            pl.BlockSpec((SEG, D), lambda a: (a, 0)),
            pl.BlockSpec((SEG, D), lambda a: (a, 0)),
        ],
        out_shape=[
            jax.ShapeDtypeStruct((STRIDE, D), jnp.float32),
            jax.ShapeDtypeStruct((STRIDE, D), jnp.float32),
        ],
        scratch_shapes=[
            pltpu.VMEM((SEG, HALF), jnp.float32),
            pltpu.VMEM((SEG, HALF), jnp.float32),
        ],
    )()


def kernel(cls_embeddings, token_ids, token_table, type_table, W, b):
    ids = token_ids.reshape(-1).astype(jnp.int32)
    cls2d = cls_embeddings.reshape(BATCH * SEG, D).astype(jnp.float32)
    cls_res = _cls_proj(cls2d, W, b.reshape(1, D),
                        type_table[0].reshape(1, D))
    rope_cos, rope_sin = _rope()
    out = _sc_gather()(ids, token_table, type_table)
    dec = out.reshape(BATCH, STRIDE, D)
    dec = dec.at[:, :SEG, :].set(cls_res.reshape(BATCH, SEG, D))
    return dec, rope_cos, rope_sin
